# Initial kernel scaffold; baseline (speedup 1.0000x reference)
import functools

import jax
import jax.numpy as jnp
from jax import lax
from jax.experimental import pallas as pl
from jax.experimental.pallas import tpu as pltpu
from jax.experimental.pallas import tpu_sc as plsc

N_NODES = 10000
N_EDGES = 320000
D = 128

NUM_CORES = 2
NUM_SUBCORES = 16
CHUNK = 128
N_CHUNKS = N_EDGES // CHUNK
CHUNKS_PER_CORE = N_CHUNKS // NUM_CORES
CHUNK_BASE = CHUNKS_PER_CORE // NUM_SUBCORES
CHUNK_REM = CHUNKS_PER_CORE % NUM_SUBCORES
ROWS_PER_SUBCORE = N_NODES // NUM_SUBCORES

_MESH = plsc.VectorSubcoreMesh(core_axis_name="c", subcore_axis_name="s")


def _chunk_range():
    c = lax.axis_index("c")
    s = lax.axis_index("s")
    start = c * CHUNKS_PER_CORE + s * CHUNK_BASE + jnp.minimum(s, CHUNK_REM)
    n = CHUNK_BASE + jnp.where(s < CHUNK_REM, 1, 0)
    return start, n


def _sc_degree(dst2d):
    @functools.partial(
        pl.kernel,
        out_type=jax.ShapeDtypeStruct((NUM_CORES, N_NODES, 16), jnp.float32),
        mesh=_MESH,
        scratch_types=[
            pltpu.VMEM((1, CHUNK), jnp.int32),
            pltpu.VMEM((CHUNK, 16), jnp.float32),
            pltpu.VMEM((ROWS_PER_SUBCORE, 16), jnp.float32),
            pltpu.VMEM_SHARED((N_NODES, 16), jnp.float32),
        ],
    )
    def deg_kernel(dst_hbm, out_hbm, idx_v, ones_v, zero_v, acc_sh):
        c = lax.axis_index("c")
        s = lax.axis_index("s")

        @pl.loop(0, CHUNK)
        def _(i):
            ones_v[i, :] = jnp.ones((16,), jnp.float32)

        @pl.loop(0, ROWS_PER_SUBCORE)
        def _(i):
            zero_v[i, :] = jnp.zeros((16,), jnp.float32)

        pltpu.sync_copy(zero_v, acc_sh.at[pl.ds(s * ROWS_PER_SUBCORE, ROWS_PER_SUBCORE)])
        plsc.subcore_barrier()

        start, n = _chunk_range()

        def body(j, _):
            pltpu.sync_copy(dst_hbm.at[start + j], idx_v.at[0])
            pltpu.sync_copy(ones_v, acc_sh.at[idx_v.at[0]], add=True)
            return 0

        lax.fori_loop(0, n, body, 0)
        plsc.subcore_barrier()

        pltpu.sync_copy(
            acc_sh.at[pl.ds(s * ROWS_PER_SUBCORE, ROWS_PER_SUBCORE)],
            out_hbm.at[c, pl.ds(s * ROWS_PER_SUBCORE, ROWS_PER_SUBCORE)],
        )

    return deg_kernel(dst2d)


def _sc_scatter(u, src2d, dst2d):
    @functools.partial(
        pl.kernel,
        out_type=jax.ShapeDtypeStruct((NUM_CORES, N_NODES, D), jnp.float32),
        mesh=_MESH,
        scratch_types=[
            pltpu.VMEM((1, CHUNK), jnp.int32),
            pltpu.VMEM((1, CHUNK), jnp.int32),
            pltpu.VMEM((CHUNK, D), jnp.float32),
            pltpu.VMEM_SHARED((N_NODES, D), jnp.float32),
        ],
    )
    def scatter_kernel(u_hbm, src_hbm, dst_hbm, out_hbm, sidx_v, didx_v, rows_v, acc_sh):
        c = lax.axis_index("c")
        s = lax.axis_index("s")

        @pl.loop(0, CHUNK)
        def _(i):
            @pl.loop(0, D, step=16)
            def _(j):
                rows_v[i, pl.ds(j, 16)] = jnp.zeros((16,), jnp.float32)

        @pl.loop(0, 5)
        def _(k):
            pltpu.sync_copy(
                rows_v.at[pl.ds(0, 125)],
                acc_sh.at[pl.ds(s * ROWS_PER_SUBCORE + k * 125, 125)],
            )

        plsc.subcore_barrier()

        start, n = _chunk_range()

        def body(j, _):
            pltpu.sync_copy(src_hbm.at[start + j], sidx_v.at[0])
            pltpu.sync_copy(dst_hbm.at[start + j], didx_v.at[0])
            pltpu.sync_copy(u_hbm.at[sidx_v.at[0]], rows_v)
            pltpu.sync_copy(rows_v, acc_sh.at[didx_v.at[0]], add=True)
            return 0

        lax.fori_loop(0, n, body, 0)
        plsc.subcore_barrier()

        pltpu.sync_copy(
            acc_sh.at[pl.ds(s * ROWS_PER_SUBCORE, ROWS_PER_SUBCORE)],
            out_hbm.at[c, pl.ds(s * ROWS_PER_SUBCORE, ROWS_PER_SUBCORE)],
        )

    return scatter_kernel(u, src2d, dst2d)


_BR = 1000


def _deg_block_spec():
    return pl.BlockSpec((NUM_CORES, _BR, 16), lambda i: (0, i, 0))


def _d_from_deg(deg_ref):
    deg = deg_ref[0, :, 0:1] + deg_ref[1, :, 0:1] + 1.0
    return lax.rsqrt(deg)


def _tc_layer_in(x, W, degp):
    def body(deg_ref, x_ref, w_ref, u_ref):
        d = _d_from_deg(deg_ref)
        u_ref[...] = (
            jnp.dot(x_ref[...], w_ref[...], preferred_element_type=jnp.float32) * d
        )

    return pl.pallas_call(
        body,
        grid=(N_NODES // _BR,),
        in_specs=[
            _deg_block_spec(),
            pl.BlockSpec((_BR, D), lambda i: (i, 0)),
            pl.BlockSpec((D, D), lambda i: (0, 0)),
        ],
        out_specs=pl.BlockSpec((_BR, D), lambda i: (i, 0)),
        out_shape=jax.ShapeDtypeStruct((N_NODES, D), jnp.float32),
    )(degp, x, W)


def _tc_mid(acc, u1, b1, a, W2, degp):
    def body(deg_ref, acc_ref, u1_ref, b1_ref, a_ref, w2_ref, u2_ref):
        d = _d_from_deg(deg_ref)
        t = (acc_ref[0] + acc_ref[1] + u1_ref[...]) * d + b1_ref[...]
        t = jnp.where(t >= 0.0, t, a_ref[...] * t)
        u2_ref[...] = (
            jnp.dot(t, w2_ref[...], preferred_element_type=jnp.float32) * d
        )

    return pl.pallas_call(
        body,
        grid=(N_NODES // _BR,),
        in_specs=[
            _deg_block_spec(),
            pl.BlockSpec((NUM_CORES, _BR, D), lambda i: (0, i, 0)),
            pl.BlockSpec((_BR, D), lambda i: (i, 0)),
            pl.BlockSpec((1, D), lambda i: (0, 0)),
            pl.BlockSpec((1, D), lambda i: (0, 0)),
            pl.BlockSpec((D, D), lambda i: (0, 0)),
        ],
        out_specs=pl.BlockSpec((_BR, D), lambda i: (i, 0)),
        out_shape=jax.ShapeDtypeStruct((N_NODES, D), jnp.float32),
    )(degp, acc, u1, b1, a, W2)


def _tc_final(acc, u2, b2, a, degp):
    def body(deg_ref, acc_ref, u2_ref, b2_ref, a_ref, o_ref):
        d = _d_from_deg(deg_ref)
        t = (acc_ref[0] + acc_ref[1] + u2_ref[...]) * d + b2_ref[...]
        o_ref[...] = jnp.where(t >= 0.0, t, a_ref[...] * t)

    return pl.pallas_call(
        body,
        grid=(N_NODES // _BR,),
        in_specs=[
            _deg_block_spec(),
            pl.BlockSpec((NUM_CORES, _BR, D), lambda i: (0, i, 0)),
            pl.BlockSpec((_BR, D), lambda i: (i, 0)),
            pl.BlockSpec((1, D), lambda i: (0, 0)),
            pl.BlockSpec((1, D), lambda i: (0, 0)),
        ],
        out_specs=pl.BlockSpec((_BR, D), lambda i: (i, 0)),
        out_shape=jax.ShapeDtypeStruct((N_NODES, D), jnp.float32),
    )(degp, acc, u2, b2, a)


def kernel(x, edge_index, W1, b1, W2, b2, a):
    src2d = edge_index[0].astype(jnp.int32).reshape(N_CHUNKS, CHUNK)
    dst2d = edge_index[1].astype(jnp.int32).reshape(N_CHUNKS, CHUNK)
    b1r = b1.reshape(1, D)
    b2r = b2.reshape(1, D)
    ar = a.reshape(1, D)

    degp = _sc_degree(dst2d)
    u1 = _tc_layer_in(x, W1, degp)
    acc1 = _sc_scatter(u1, src2d, dst2d)
    u2 = _tc_mid(acc1, u1, b1r, ar, W2, degp)
    acc2 = _sc_scatter(u2, src2d, dst2d)
    return _tc_final(acc2, u2, b2r, ar, degp)


# speedup vs baseline: 19.2883x; 19.2883x over previous
import functools

import jax
import jax.numpy as jnp
from jax import lax
from jax.experimental import pallas as pl
from jax.experimental.pallas import tpu as pltpu
from jax.experimental.pallas import tpu_sc as plsc

N_NODES = 10000
N_EDGES = 320000
D = 128

NUM_CORES = 2
NUM_SUBCORES = 16
NUM_TILES = NUM_CORES * NUM_SUBCORES
CHUNK = 128
GROUP = 8
N_CHUNKS_PAD = 2560
N_EDGES_PAD = N_CHUNKS_PAD * CHUNK
CHUNKS_PER_TILE = N_CHUNKS_PAD // NUM_TILES
GROUPS_PER_TILE = CHUNKS_PER_TILE // GROUP
N_PAD = 10240
ROWS_PER_SUBCORE = N_PAD // NUM_SUBCORES
ZERO_SLABS = ROWS_PER_SUBCORE // CHUNK

_MESH = plsc.VectorSubcoreMesh(core_axis_name="c", subcore_axis_name="s")


def _zero_rows_buffer(rows_v):
    @pl.loop(0, CHUNK)
    def _(i):
        @pl.loop(0, D, step=16)
        def _(j):
            rows_v[i, pl.ds(j, 16)] = jnp.zeros((16,), jnp.float32)


def _zero_accumulator(s, rows_v, acc_sh):
    for i in range(NUM_SUBCORES):
        @pl.when(s == i)
        def _():
            for k in range(ZERO_SLABS):
                pltpu.sync_copy(
                    rows_v,
                    acc_sh.at[pl.ds(i * ROWS_PER_SUBCORE + k * CHUNK, CHUNK)],
                )


def _copy_out_accumulator(c, s, acc_sh, out_hbm):
    for i in range(NUM_SUBCORES):
        @pl.when(s == i)
        def _():
            pltpu.sync_copy(
                acc_sh.at[pl.ds(i * ROWS_PER_SUBCORE, ROWS_PER_SUBCORE)],
                out_hbm.at[c, pl.ds(i * ROWS_PER_SUBCORE, ROWS_PER_SUBCORE)],
            )


def _sc_degree(dst2d):
    @functools.partial(
        pl.kernel,
        out_type=jax.ShapeDtypeStruct((NUM_CORES, N_PAD, D), jnp.float32),
        mesh=_MESH,
        scratch_types=[
            pltpu.VMEM((GROUP, CHUNK), jnp.int32),
            pltpu.VMEM((CHUNK, D), jnp.float32),
            pltpu.VMEM_SHARED((N_PAD, D), jnp.float32),
        ],
    )
    def deg_kernel(dst_hbm, out_hbm, idx_v, ones_v, acc_sh):
        c = lax.axis_index("c")
        s = lax.axis_index("s")
        w = c * NUM_SUBCORES + s

        _zero_rows_buffer(ones_v)
        _zero_accumulator(s, ones_v, acc_sh)

        @pl.loop(0, CHUNK)
        def _(i):
            @pl.loop(0, D, step=16)
            def _(j):
                ones_v[i, pl.ds(j, 16)] = jnp.ones((16,), jnp.float32)

        plsc.subcore_barrier()

        def body(g, carry):
            base = pl.multiple_of(w * CHUNKS_PER_TILE + g * GROUP, GROUP)
            pltpu.sync_copy(dst_hbm.at[pl.ds(base, GROUP)], idx_v)
            for k in range(GROUP):
                pltpu.sync_copy(ones_v, acc_sh.at[idx_v.at[k]], add=True)
            return carry

        lax.fori_loop(0, GROUPS_PER_TILE, body, 0)
        plsc.subcore_barrier()
        _copy_out_accumulator(c, s, acc_sh, out_hbm)

    return deg_kernel(dst2d)


def _sc_scatter(u, src2d, dst2d):
    @functools.partial(
        pl.kernel,
        out_type=jax.ShapeDtypeStruct((NUM_CORES, N_PAD, D), jnp.float32),
        mesh=_MESH,
        scratch_types=[
            pltpu.VMEM((GROUP, CHUNK), jnp.int32),
            pltpu.VMEM((GROUP, CHUNK), jnp.int32),
            pltpu.VMEM((CHUNK, D), jnp.float32),
            pltpu.VMEM_SHARED((N_PAD, D), jnp.float32),
        ],
    )
    def scatter_kernel(u_hbm, src_hbm, dst_hbm, out_hbm, sidx_v, didx_v, rows_v, acc_sh):
        c = lax.axis_index("c")
        s = lax.axis_index("s")
        w = c * NUM_SUBCORES + s

        _zero_rows_buffer(rows_v)
        _zero_accumulator(s, rows_v, acc_sh)
        plsc.subcore_barrier()

        def body(g, carry):
            base = pl.multiple_of(w * CHUNKS_PER_TILE + g * GROUP, GROUP)
            pltpu.sync_copy(src_hbm.at[pl.ds(base, GROUP)], sidx_v)
            pltpu.sync_copy(dst_hbm.at[pl.ds(base, GROUP)], didx_v)
            for k in range(GROUP):
                pltpu.sync_copy(u_hbm.at[sidx_v.at[k]], rows_v)
                pltpu.sync_copy(rows_v, acc_sh.at[didx_v.at[k]], add=True)
            return carry

        lax.fori_loop(0, GROUPS_PER_TILE, body, 0)
        plsc.subcore_barrier()
        _copy_out_accumulator(c, s, acc_sh, out_hbm)

    return scatter_kernel(u, src2d, dst2d)


_BR = 1024
_GRID = N_PAD // _BR


def _part_spec():
    return pl.BlockSpec((NUM_CORES, _BR, D), lambda i: (0, i, 0))


def _d_from_deg(deg_ref):
    return lax.rsqrt(deg_ref[0] + deg_ref[1] + 1.0)


def _tc_layer_in(x, W, degp):
    def body(deg_ref, x_ref, w_ref, u_ref):
        d = _d_from_deg(deg_ref)
        u_ref[...] = (
            jnp.dot(x_ref[...], w_ref[...], preferred_element_type=jnp.float32) * d
        )

    return pl.pallas_call(
        body,
        grid=(_GRID,),
        in_specs=[
            _part_spec(),
            pl.BlockSpec((_BR, D), lambda i: (i, 0)),
            pl.BlockSpec((D, D), lambda i: (0, 0)),
        ],
        out_specs=pl.BlockSpec((_BR, D), lambda i: (i, 0)),
        out_shape=jax.ShapeDtypeStruct((N_PAD, D), jnp.float32),
    )(degp, x, W)


def _tc_mid(acc, u1, b1, a, W2, degp):
    def body(deg_ref, acc_ref, u1_ref, b1_ref, a_ref, w2_ref, u2_ref):
        d = _d_from_deg(deg_ref)
        t = (acc_ref[0] + acc_ref[1] + u1_ref[...]) * d + b1_ref[...]
        t = jnp.where(t >= 0.0, t, a_ref[...] * t)
        u2_ref[...] = (
            jnp.dot(t, w2_ref[...], preferred_element_type=jnp.float32) * d
        )

    return pl.pallas_call(
        body,
        grid=(_GRID,),
        in_specs=[
            _part_spec(),
            _part_spec(),
            pl.BlockSpec((_BR, D), lambda i: (i, 0)),
            pl.BlockSpec((1, D), lambda i: (0, 0)),
            pl.BlockSpec((1, D), lambda i: (0, 0)),
            pl.BlockSpec((D, D), lambda i: (0, 0)),
        ],
        out_specs=pl.BlockSpec((_BR, D), lambda i: (i, 0)),
        out_shape=jax.ShapeDtypeStruct((N_PAD, D), jnp.float32),
    )(degp, acc, u1, b1, a, W2)


def _tc_final(acc, u2, b2, a, degp):
    def body(deg_ref, acc_ref, u2_ref, b2_ref, a_ref, o_ref):
        d = _d_from_deg(deg_ref)
        t = (acc_ref[0] + acc_ref[1] + u2_ref[...]) * d + b2_ref[...]
        o_ref[...] = jnp.where(t >= 0.0, t, a_ref[...] * t)

    return pl.pallas_call(
        body,
        grid=(_GRID,),
        in_specs=[
            _part_spec(),
            _part_spec(),
            pl.BlockSpec((_BR, D), lambda i: (i, 0)),
            pl.BlockSpec((1, D), lambda i: (0, 0)),
            pl.BlockSpec((1, D), lambda i: (0, 0)),
        ],
        out_specs=pl.BlockSpec((_BR, D), lambda i: (i, 0)),
        out_shape=jax.ShapeDtypeStruct((N_PAD, D), jnp.float32),
    )(degp, acc, u2, b2, a)


def kernel(x, edge_index, W1, b1, W2, b2, a):
    src = edge_index[0].astype(jnp.int32)
    dst = edge_index[1].astype(jnp.int32)
    n_extra = N_EDGES_PAD - N_EDGES
    src_pad = jnp.arange(n_extra, dtype=jnp.int32) % N_NODES
    dst_pad = N_NODES + jnp.arange(n_extra, dtype=jnp.int32) % (N_PAD - N_NODES)
    src2d = jnp.concatenate([src, src_pad]).reshape(N_CHUNKS_PAD, CHUNK)
    dst2d = jnp.concatenate([dst, dst_pad]).reshape(N_CHUNKS_PAD, CHUNK)
    xp = jnp.pad(x, ((0, N_PAD - N_NODES), (0, 0)))
    b1r = b1.reshape(1, D)
    b2r = b2.reshape(1, D)
    ar = a.reshape(1, D)

    degp = _sc_degree(dst2d)
    u1 = _tc_layer_in(xp, W1, degp)
    acc1 = _sc_scatter(u1, src2d, dst2d)
    u2 = _tc_mid(acc1, u1, b1r, ar, W2, degp)
    acc2 = _sc_scatter(u2, src2d, dst2d)
    return _tc_final(acc2, u2, b2r, ar, degp)[:N_NODES]


# speedup vs baseline: 24.9442x; 1.2932x over previous
import functools

import jax
import jax.numpy as jnp
from jax import lax
from jax.experimental import pallas as pl
from jax.experimental.pallas import tpu as pltpu
from jax.experimental.pallas import tpu_sc as plsc

N_NODES = 10000
N_EDGES = 320000
D = 128

NUM_CORES = 2
NUM_SUBCORES = 16
NUM_TILES = NUM_CORES * NUM_SUBCORES
CHUNK = 128
GROUP = 8
N_CHUNKS_PAD = 2560
N_EDGES_PAD = N_CHUNKS_PAD * CHUNK
CHUNKS_PER_TILE = N_CHUNKS_PAD // NUM_TILES
GROUPS_PER_TILE = CHUNKS_PER_TILE // GROUP
NBUF = 2
N_PAD = 10240
ROWS_PER_SUBCORE = N_PAD // NUM_SUBCORES
ZERO_SLABS = ROWS_PER_SUBCORE // CHUNK

_MESH = plsc.VectorSubcoreMesh(core_axis_name="c", subcore_axis_name="s")


def _zero_rows_buffer(rows_v):
    @pl.loop(0, CHUNK)
    def _(i):
        @pl.loop(0, D, step=16)
        def _(j):
            rows_v[i, pl.ds(j, 16)] = jnp.zeros((16,), jnp.float32)


def _zero_accumulator(s, rows_v, acc_sh):
    for i in range(NUM_SUBCORES):
        @pl.when(s == i)
        def _():
            for k in range(ZERO_SLABS):
                pltpu.sync_copy(
                    rows_v,
                    acc_sh.at[pl.ds(i * ROWS_PER_SUBCORE + k * CHUNK, CHUNK)],
                )


def _copy_out_accumulator(c, s, acc_sh, out_hbm):
    for i in range(NUM_SUBCORES):
        @pl.when(s == i)
        def _():
            pltpu.sync_copy(
                acc_sh.at[pl.ds(i * ROWS_PER_SUBCORE, ROWS_PER_SUBCORE)],
                out_hbm.at[c, pl.ds(i * ROWS_PER_SUBCORE, ROWS_PER_SUBCORE)],
            )


def _sc_degree(dst2d):
    @functools.partial(
        pl.kernel,
        out_type=jax.ShapeDtypeStruct((NUM_CORES, N_PAD, D), jnp.float32),
        mesh=_MESH,
        scratch_types=[
            pltpu.VMEM((CHUNKS_PER_TILE, CHUNK), jnp.int32),
            pltpu.VMEM((CHUNK, D), jnp.float32),
            pltpu.VMEM_SHARED((N_PAD, D), jnp.float32),
            pltpu.SemaphoreType.DMA,
        ],
    )
    def deg_kernel(dst_hbm, out_hbm, idx_v, ones_v, acc_sh, sem):
        c = lax.axis_index("c")
        s = lax.axis_index("s")
        w = c * NUM_SUBCORES + s

        _zero_rows_buffer(ones_v)
        _zero_accumulator(s, ones_v, acc_sh)

        @pl.loop(0, CHUNK)
        def _(i):
            @pl.loop(0, D, step=16)
            def _(j):
                ones_v[i, pl.ds(j, 16)] = jnp.ones((16,), jnp.float32)

        base = pl.multiple_of(w * CHUNKS_PER_TILE, 8)
        pltpu.sync_copy(dst_hbm.at[pl.ds(base, CHUNKS_PER_TILE)], idx_v)
        plsc.subcore_barrier()

        def body(g, carry):
            handles = [
                pltpu.async_copy(
                    ones_v, acc_sh.at[idx_v.at[g * GROUP + k]], sem, add=True
                )
                for k in range(GROUP)
            ]
            for h in handles:
                h.wait()
            return carry

        lax.fori_loop(0, GROUPS_PER_TILE, body, 0)
        plsc.subcore_barrier()
        _copy_out_accumulator(c, s, acc_sh, out_hbm)

    return deg_kernel(dst2d)


def _sc_scatter(u, src2d, dst2d):
    @functools.partial(
        pl.kernel,
        out_type=jax.ShapeDtypeStruct((NUM_CORES, N_PAD, D), jnp.float32),
        mesh=_MESH,
        scratch_types=[
            pltpu.VMEM((GROUP, CHUNK), jnp.int32),
            pltpu.VMEM((GROUP, CHUNK), jnp.int32),
            pltpu.VMEM((NBUF * CHUNK, D), jnp.float32),
            pltpu.VMEM_SHARED((N_PAD, D), jnp.float32),
        ]
        + [pltpu.SemaphoreType.DMA] * (2 * NBUF),
    )
    def scatter_kernel(
        u_hbm, src_hbm, dst_hbm, out_hbm, sidx_v, didx_v, rows_v, acc_sh, *sems
    ):
        c = lax.axis_index("c")
        s = lax.axis_index("s")
        w = c * NUM_SUBCORES + s
        gsem, ssem = sems[:NBUF], sems[NBUF:]

        def buf(b):
            return rows_v.at[pl.ds(b * CHUNK, CHUNK)]

        _zero_rows_buffer(buf(0))
        _zero_accumulator(s, buf(0), acc_sh)
        plsc.subcore_barrier()

        def gather(k, b):
            return pltpu.async_copy(u_hbm.at[sidx_v.at[k]], buf(b), gsem[b])

        def body(g, carry):
            base = pl.multiple_of(w * CHUNKS_PER_TILE + g * GROUP, GROUP)
            pltpu.sync_copy(src_hbm.at[pl.ds(base, GROUP)], sidx_v)
            pltpu.sync_copy(dst_hbm.at[pl.ds(base, GROUP)], didx_v)

            gathers = [None] * NBUF
            scatters = [None] * NBUF
            gathers[0] = gather(0, 0)
            for k in range(1, GROUP):
                b = k % NBUF
                ob = (k - 1) % NBUF
                if scatters[b] is not None:
                    scatters[b].wait()
                gathers[b] = gather(k, b)
                gathers[ob].wait()
                scatters[ob] = pltpu.async_copy(
                    buf(ob), acc_sh.at[didx_v.at[k - 1]], ssem[ob], add=True
                )
            last = (GROUP - 1) % NBUF
            gathers[last].wait()
            scatters[last] = pltpu.async_copy(
                buf(last), acc_sh.at[didx_v.at[GROUP - 1]], ssem[last], add=True
            )
            for b in range(NBUF):
                if scatters[b] is not None:
                    scatters[b].wait()
            return carry

        lax.fori_loop(0, GROUPS_PER_TILE, body, 0)
        plsc.subcore_barrier()
        _copy_out_accumulator(c, s, acc_sh, out_hbm)

    return scatter_kernel(u, src2d, dst2d)


_BR = 1024
_GRID = N_PAD // _BR


def _part_spec():
    return pl.BlockSpec((NUM_CORES, _BR, D), lambda i: (0, i, 0))


def _d_from_deg(deg_ref):
    return lax.rsqrt(deg_ref[0] + deg_ref[1] + 1.0)


def _tc_layer_in(x, W, degp):
    def body(deg_ref, x_ref, w_ref, u_ref):
        d = _d_from_deg(deg_ref)
        u_ref[...] = (
            jnp.dot(x_ref[...], w_ref[...], preferred_element_type=jnp.float32) * d
        )

    return pl.pallas_call(
        body,
        grid=(_GRID,),
        in_specs=[
            _part_spec(),
            pl.BlockSpec((_BR, D), lambda i: (i, 0)),
            pl.BlockSpec((D, D), lambda i: (0, 0)),
        ],
        out_specs=pl.BlockSpec((_BR, D), lambda i: (i, 0)),
        out_shape=jax.ShapeDtypeStruct((N_PAD, D), jnp.float32),
    )(degp, x, W)


def _tc_mid(acc, u1, b1, a, W2, degp):
    def body(deg_ref, acc_ref, u1_ref, b1_ref, a_ref, w2_ref, u2_ref):
        d = _d_from_deg(deg_ref)
        t = (acc_ref[0] + acc_ref[1] + u1_ref[...]) * d + b1_ref[...]
        t = jnp.where(t >= 0.0, t, a_ref[...] * t)
        u2_ref[...] = (
            jnp.dot(t, w2_ref[...], preferred_element_type=jnp.float32) * d
        )

    return pl.pallas_call(
        body,
        grid=(_GRID,),
        in_specs=[
            _part_spec(),
            _part_spec(),
            pl.BlockSpec((_BR, D), lambda i: (i, 0)),
            pl.BlockSpec((1, D), lambda i: (0, 0)),
            pl.BlockSpec((1, D), lambda i: (0, 0)),
            pl.BlockSpec((D, D), lambda i: (0, 0)),
        ],
        out_specs=pl.BlockSpec((_BR, D), lambda i: (i, 0)),
        out_shape=jax.ShapeDtypeStruct((N_PAD, D), jnp.float32),
    )(degp, acc, u1, b1, a, W2)


def _tc_final(acc, u2, b2, a, degp):
    def body(deg_ref, acc_ref, u2_ref, b2_ref, a_ref, o_ref):
        d = _d_from_deg(deg_ref)
        t = (acc_ref[0] + acc_ref[1] + u2_ref[...]) * d + b2_ref[...]
        o_ref[...] = jnp.where(t >= 0.0, t, a_ref[...] * t)

    return pl.pallas_call(
        body,
        grid=(_GRID,),
        in_specs=[
            _part_spec(),
            _part_spec(),
            pl.BlockSpec((_BR, D), lambda i: (i, 0)),
            pl.BlockSpec((1, D), lambda i: (0, 0)),
            pl.BlockSpec((1, D), lambda i: (0, 0)),
        ],
        out_specs=pl.BlockSpec((_BR, D), lambda i: (i, 0)),
        out_shape=jax.ShapeDtypeStruct((N_PAD, D), jnp.float32),
    )(degp, acc, u2, b2, a)


def kernel(x, edge_index, W1, b1, W2, b2, a):
    src = edge_index[0].astype(jnp.int32)
    dst = edge_index[1].astype(jnp.int32)
    n_extra = N_EDGES_PAD - N_EDGES
    src_pad = jnp.arange(n_extra, dtype=jnp.int32) % N_NODES
    dst_pad = N_NODES + jnp.arange(n_extra, dtype=jnp.int32) % (N_PAD - N_NODES)
    src2d = jnp.concatenate([src, src_pad]).reshape(N_CHUNKS_PAD, CHUNK)
    dst2d = jnp.concatenate([dst, dst_pad]).reshape(N_CHUNKS_PAD, CHUNK)
    xp = jnp.pad(x, ((0, N_PAD - N_NODES), (0, 0)))
    b1r = b1.reshape(1, D)
    b2r = b2.reshape(1, D)
    ar = a.reshape(1, D)

    degp = _sc_degree(dst2d)
    u1 = _tc_layer_in(xp, W1, degp)
    acc1 = _sc_scatter(u1, src2d, dst2d)
    u2 = _tc_mid(acc1, u1, b1r, ar, W2, degp)
    acc2 = _sc_scatter(u2, src2d, dst2d)
    return _tc_final(acc2, u2, b2r, ar, degp)[:N_NODES]


# speedup vs baseline: 26.2650x; 1.0530x over previous
import functools

import jax
import jax.numpy as jnp
from jax import lax
from jax.experimental import pallas as pl
from jax.experimental.pallas import tpu as pltpu
from jax.experimental.pallas import tpu_sc as plsc

N_NODES = 10000
N_EDGES = 320000
D = 128

NUM_CORES = 2
NUM_SUBCORES = 16
NUM_TILES = NUM_CORES * NUM_SUBCORES
CHUNK = 128
GROUP = 8
N_CHUNKS_PAD = 2560
N_EDGES_PAD = N_CHUNKS_PAD * CHUNK
CHUNKS_PER_TILE = N_CHUNKS_PAD // NUM_TILES
GROUPS_PER_TILE = CHUNKS_PER_TILE // GROUP
NBUF = 2
N_PAD = 10240
ROWS_PER_SUBCORE = N_PAD // NUM_SUBCORES
ZERO_SLABS = ROWS_PER_SUBCORE // CHUNK

_MESH = plsc.VectorSubcoreMesh(core_axis_name="c", subcore_axis_name="s")


def _zero_rows_buffer(rows_v):
    @pl.loop(0, CHUNK)
    def _(i):
        @pl.loop(0, D, step=16)
        def _(j):
            rows_v[i, pl.ds(j, 16)] = jnp.zeros((16,), jnp.float32)


def _zero_accumulator(s, rows_v, acc_sh):
    for i in range(NUM_SUBCORES):
        @pl.when(s == i)
        def _():
            for k in range(ZERO_SLABS):
                pltpu.sync_copy(
                    rows_v,
                    acc_sh.at[pl.ds(i * ROWS_PER_SUBCORE + k * CHUNK, CHUNK)],
                )


def _copy_out_accumulator(c, s, acc_sh, out_hbm):
    for i in range(NUM_SUBCORES):
        @pl.when(s == i)
        def _():
            pltpu.sync_copy(
                acc_sh.at[pl.ds(i * ROWS_PER_SUBCORE, ROWS_PER_SUBCORE)],
                out_hbm.at[c, pl.ds(i * ROWS_PER_SUBCORE, ROWS_PER_SUBCORE)],
            )


def _sc_degree(dst2d):
    @functools.partial(
        pl.kernel,
        out_type=jax.ShapeDtypeStruct((NUM_CORES, N_PAD, D), jnp.float32),
        mesh=_MESH,
        scratch_types=[
            pltpu.VMEM((CHUNKS_PER_TILE, CHUNK), jnp.int32),
            pltpu.VMEM((CHUNK, D), jnp.float32),
            pltpu.VMEM_SHARED((N_PAD, D), jnp.float32),
            pltpu.SemaphoreType.DMA,
        ],
    )
    def deg_kernel(dst_hbm, out_hbm, idx_v, ones_v, acc_sh, sem):
        c = lax.axis_index("c")
        s = lax.axis_index("s")
        w = c * NUM_SUBCORES + s

        _zero_rows_buffer(ones_v)
        _zero_accumulator(s, ones_v, acc_sh)

        @pl.loop(0, CHUNK)
        def _(i):
            @pl.loop(0, D, step=16)
            def _(j):
                ones_v[i, pl.ds(j, 16)] = jnp.ones((16,), jnp.float32)

        base = pl.multiple_of(w * CHUNKS_PER_TILE, 8)
        pltpu.sync_copy(dst_hbm.at[pl.ds(base, CHUNKS_PER_TILE)], idx_v)
        plsc.subcore_barrier()

        def scat(t):
            return pltpu.async_copy(ones_v, acc_sh.at[idx_v.at[t]], sem, add=True)

        def drain(t):
            pltpu.make_async_copy(ones_v, acc_sh.at[idx_v.at[t]], sem).wait()

        for t in range(GROUP):
            scat(t)

        def body(t, carry):
            drain(t - GROUP)
            scat(t)
            return carry

        lax.fori_loop(GROUP, CHUNKS_PER_TILE, body, 0)
        for t in range(CHUNKS_PER_TILE - GROUP, CHUNKS_PER_TILE):
            drain(t)
        plsc.subcore_barrier()
        _copy_out_accumulator(c, s, acc_sh, out_hbm)

    return deg_kernel(dst2d)


def _sc_scatter(u, src2d, dst2d):
    @functools.partial(
        pl.kernel,
        out_type=jax.ShapeDtypeStruct((NUM_CORES, N_PAD, D), jnp.float32),
        mesh=_MESH,
        scratch_types=[
            pltpu.VMEM((GROUP, CHUNK), jnp.int32),
            pltpu.VMEM((CHUNKS_PER_TILE, CHUNK), jnp.int32),
            pltpu.VMEM((NBUF * CHUNK, D), jnp.float32),
            pltpu.VMEM_SHARED((N_PAD, D), jnp.float32),
        ]
        + [pltpu.SemaphoreType.DMA] * (2 * NBUF),
    )
    def scatter_kernel(
        u_hbm, src_hbm, dst_hbm, out_hbm, sidx_v, didx_v, rows_v, acc_sh, *sems
    ):
        c = lax.axis_index("c")
        s = lax.axis_index("s")
        w = c * NUM_SUBCORES + s
        gsem, ssem = sems[:NBUF], sems[NBUF:]

        def buf(b):
            return rows_v.at[pl.ds(b * CHUNK, CHUNK)]

        _zero_rows_buffer(buf(0))
        _zero_accumulator(s, buf(0), acc_sh)

        base = pl.multiple_of(w * CHUNKS_PER_TILE, 8)
        pltpu.sync_copy(dst_hbm.at[pl.ds(base, CHUNKS_PER_TILE)], didx_v)
        plsc.subcore_barrier()

        def load_src_group(g):
            gb = pl.multiple_of(w * CHUNKS_PER_TILE + g * GROUP, GROUP)
            pltpu.sync_copy(src_hbm.at[pl.ds(gb, GROUP)], sidx_v)

        def gather(t, k, b):
            del t
            return pltpu.async_copy(u_hbm.at[sidx_v.at[k]], buf(b), gsem[b])

        def scatter(t, b):
            return pltpu.async_copy(buf(b), acc_sh.at[didx_v.at[t]], ssem[b], add=True)

        def wait_gather(b):
            pltpu.make_async_copy(u_hbm.at[sidx_v.at[0]], buf(b), gsem[b]).wait()

        def wait_scatter(t, b):
            pltpu.make_async_copy(buf(b), acc_sh.at[didx_v.at[t]], ssem[b]).wait()

        load_src_group(0)
        for k in range(GROUP):
            b = k % NBUF
            if k >= 2:
                wait_scatter(k - 2, b)
            gather(k, k, b)
            if k >= 1:
                ob = (k - 1) % NBUF
                wait_gather(ob)
                scatter(k - 1, ob)

        def body(g, carry):
            t0 = g * GROUP
            lb = (GROUP - 1) % NBUF
            wait_gather(lb)
            scatter(t0 - 1, lb)
            load_src_group(g)
            for k in range(GROUP):
                b = k % NBUF
                wait_scatter(t0 + k - 2, b)
                gather(t0 + k, k, b)
                if k >= 1:
                    ob = (k - 1) % NBUF
                    wait_gather(ob)
                    scatter(t0 + k - 1, ob)
            return carry

        lax.fori_loop(1, GROUPS_PER_TILE, body, 0)

        last_t = CHUNKS_PER_TILE - 1
        lb = (GROUP - 1) % NBUF
        wait_gather(lb)
        scatter(last_t, lb)
        wait_scatter(last_t - 1, (last_t - 1) % NBUF)
        wait_scatter(last_t, lb)

        plsc.subcore_barrier()
        _copy_out_accumulator(c, s, acc_sh, out_hbm)

    return scatter_kernel(u, src2d, dst2d)


_BR = 1024
_GRID = N_PAD // _BR


def _part_spec():
    return pl.BlockSpec((NUM_CORES, _BR, D), lambda i: (0, i, 0))


def _d_from_deg(deg_ref):
    return lax.rsqrt(deg_ref[0] + deg_ref[1] + 1.0)


def _tc_layer_in(x, W, degp):
    def body(deg_ref, x_ref, w_ref, u_ref):
        d = _d_from_deg(deg_ref)
        u_ref[...] = (
            jnp.dot(x_ref[...], w_ref[...], preferred_element_type=jnp.float32) * d
        )

    return pl.pallas_call(
        body,
        grid=(_GRID,),
        in_specs=[
            _part_spec(),
            pl.BlockSpec((_BR, D), lambda i: (i, 0)),
            pl.BlockSpec((D, D), lambda i: (0, 0)),
        ],
        out_specs=pl.BlockSpec((_BR, D), lambda i: (i, 0)),
        out_shape=jax.ShapeDtypeStruct((N_PAD, D), jnp.float32),
    )(degp, x, W)


def _tc_mid(acc, u1, b1, a, W2, degp):
    def body(deg_ref, acc_ref, u1_ref, b1_ref, a_ref, w2_ref, u2_ref):
        d = _d_from_deg(deg_ref)
        t = (acc_ref[0] + acc_ref[1] + u1_ref[...]) * d + b1_ref[...]
        t = jnp.where(t >= 0.0, t, a_ref[...] * t)
        u2_ref[...] = (
            jnp.dot(t, w2_ref[...], preferred_element_type=jnp.float32) * d
        )

    return pl.pallas_call(
        body,
        grid=(_GRID,),
        in_specs=[
            _part_spec(),
            _part_spec(),
            pl.BlockSpec((_BR, D), lambda i: (i, 0)),
            pl.BlockSpec((1, D), lambda i: (0, 0)),
            pl.BlockSpec((1, D), lambda i: (0, 0)),
            pl.BlockSpec((D, D), lambda i: (0, 0)),
        ],
        out_specs=pl.BlockSpec((_BR, D), lambda i: (i, 0)),
        out_shape=jax.ShapeDtypeStruct((N_PAD, D), jnp.float32),
    )(degp, acc, u1, b1, a, W2)


def _tc_final(acc, u2, b2, a, degp):
    def body(deg_ref, acc_ref, u2_ref, b2_ref, a_ref, o_ref):
        d = _d_from_deg(deg_ref)
        t = (acc_ref[0] + acc_ref[1] + u2_ref[...]) * d + b2_ref[...]
        o_ref[...] = jnp.where(t >= 0.0, t, a_ref[...] * t)

    return pl.pallas_call(
        body,
        grid=(_GRID,),
        in_specs=[
            _part_spec(),
            _part_spec(),
            pl.BlockSpec((_BR, D), lambda i: (i, 0)),
            pl.BlockSpec((1, D), lambda i: (0, 0)),
            pl.BlockSpec((1, D), lambda i: (0, 0)),
        ],
        out_specs=pl.BlockSpec((_BR, D), lambda i: (i, 0)),
        out_shape=jax.ShapeDtypeStruct((N_PAD, D), jnp.float32),
    )(degp, acc, u2, b2, a)


def kernel(x, edge_index, W1, b1, W2, b2, a):
    src = edge_index[0].astype(jnp.int32)
    dst = edge_index[1].astype(jnp.int32)
    n_extra = N_EDGES_PAD - N_EDGES
    src_pad = jnp.arange(n_extra, dtype=jnp.int32) % N_NODES
    dst_pad = N_NODES + jnp.arange(n_extra, dtype=jnp.int32) % (N_PAD - N_NODES)
    src2d = jnp.concatenate([src, src_pad]).reshape(N_CHUNKS_PAD, CHUNK)
    dst2d = jnp.concatenate([dst, dst_pad]).reshape(N_CHUNKS_PAD, CHUNK)
    xp = jnp.pad(x, ((0, N_PAD - N_NODES), (0, 0)))
    b1r = b1.reshape(1, D)
    b2r = b2.reshape(1, D)
    ar = a.reshape(1, D)

    degp = _sc_degree(dst2d)
    u1 = _tc_layer_in(xp, W1, degp)
    acc1 = _sc_scatter(u1, src2d, dst2d)
    u2 = _tc_mid(acc1, u1, b1r, ar, W2, degp)
    acc2 = _sc_scatter(u2, src2d, dst2d)
    return _tc_final(acc2, u2, b2r, ar, degp)[:N_NODES]


# speedup vs baseline: 27.1792x; 1.0348x over previous
import functools

import jax
import jax.numpy as jnp
from jax import lax
from jax.experimental import pallas as pl
from jax.experimental.pallas import tpu as pltpu
from jax.experimental.pallas import tpu_sc as plsc

N_NODES = 10000
N_EDGES = 320000
D = 128

NUM_CORES = 2
NUM_SUBCORES = 16
NUM_TILES = NUM_CORES * NUM_SUBCORES
CHUNK = 128
GROUP = 16
N_CHUNKS_PAD = 2560
N_EDGES_PAD = N_CHUNKS_PAD * CHUNK
CHUNKS_PER_TILE = N_CHUNKS_PAD // NUM_TILES
GROUPS_PER_TILE = CHUNKS_PER_TILE // GROUP
NBUF = 2
N_PAD = 10240
ROWS_PER_SUBCORE = N_PAD // NUM_SUBCORES
ZERO_SLABS = ROWS_PER_SUBCORE // CHUNK

_MESH = plsc.VectorSubcoreMesh(core_axis_name="c", subcore_axis_name="s")


def _zero_rows_buffer(rows_v):
    @pl.loop(0, CHUNK)
    def _(i):
        @pl.loop(0, D, step=16)
        def _(j):
            rows_v[i, pl.ds(j, 16)] = jnp.zeros((16,), jnp.float32)


def _zero_accumulator(s, rows_v, acc_sh):
    for i in range(NUM_SUBCORES):
        @pl.when(s == i)
        def _():
            for k in range(ZERO_SLABS):
                pltpu.sync_copy(
                    rows_v,
                    acc_sh.at[pl.ds(i * ROWS_PER_SUBCORE + k * CHUNK, CHUNK)],
                )


def _copy_out_accumulator(c, s, acc_sh, out_hbm):
    for i in range(NUM_SUBCORES):
        @pl.when(s == i)
        def _():
            pltpu.sync_copy(
                acc_sh.at[pl.ds(i * ROWS_PER_SUBCORE, ROWS_PER_SUBCORE)],
                out_hbm.at[c, pl.ds(i * ROWS_PER_SUBCORE, ROWS_PER_SUBCORE)],
            )


def _sc_degree(dst2d):
    @functools.partial(
        pl.kernel,
        out_type=jax.ShapeDtypeStruct((NUM_CORES, N_PAD, D), jnp.float32),
        mesh=_MESH,
        scratch_types=[
            pltpu.VMEM((CHUNKS_PER_TILE, CHUNK), jnp.int32),
            pltpu.VMEM((CHUNK, D), jnp.float32),
            pltpu.VMEM_SHARED((N_PAD, D), jnp.float32),
            pltpu.SemaphoreType.DMA,
        ],
    )
    def deg_kernel(dst_hbm, out_hbm, idx_v, ones_v, acc_sh, sem):
        c = lax.axis_index("c")
        s = lax.axis_index("s")
        w = c * NUM_SUBCORES + s

        _zero_rows_buffer(ones_v)
        _zero_accumulator(s, ones_v, acc_sh)

        @pl.loop(0, CHUNK)
        def _(i):
            @pl.loop(0, D, step=16)
            def _(j):
                ones_v[i, pl.ds(j, 16)] = jnp.ones((16,), jnp.float32)

        base = pl.multiple_of(w * CHUNKS_PER_TILE, 8)
        pltpu.sync_copy(dst_hbm.at[pl.ds(base, CHUNKS_PER_TILE)], idx_v)
        plsc.subcore_barrier()

        def scat(t):
            return pltpu.async_copy(ones_v, acc_sh.at[idx_v.at[t]], sem, add=True)

        def drain(t):
            pltpu.make_async_copy(ones_v, acc_sh.at[idx_v.at[t]], sem).wait()

        for t in range(GROUP):
            scat(t)

        def body(t, carry):
            drain(t - GROUP)
            scat(t)
            return carry

        lax.fori_loop(GROUP, CHUNKS_PER_TILE, body, 0)
        for t in range(CHUNKS_PER_TILE - GROUP, CHUNKS_PER_TILE):
            drain(t)
        plsc.subcore_barrier()
        _copy_out_accumulator(c, s, acc_sh, out_hbm)

    return deg_kernel(dst2d)


def _sc_scatter(u, src2d, dst2d):
    @functools.partial(
        pl.kernel,
        out_type=jax.ShapeDtypeStruct((NUM_CORES, N_PAD, D), jnp.float32),
        mesh=_MESH,
        scratch_types=[
            pltpu.VMEM((GROUP, CHUNK), jnp.int32),
            pltpu.VMEM((CHUNKS_PER_TILE, CHUNK), jnp.int32),
            pltpu.VMEM((NBUF * CHUNK, D), jnp.float32),
            pltpu.VMEM_SHARED((N_PAD, D), jnp.float32),
        ]
        + [pltpu.SemaphoreType.DMA] * (2 * NBUF),
    )
    def scatter_kernel(
        u_hbm, src_hbm, dst_hbm, out_hbm, sidx_v, didx_v, rows_v, acc_sh, *sems
    ):
        c = lax.axis_index("c")
        s = lax.axis_index("s")
        w = c * NUM_SUBCORES + s
        gsem, ssem = sems[:NBUF], sems[NBUF:]

        def buf(b):
            return rows_v.at[pl.ds(b * CHUNK, CHUNK)]

        _zero_rows_buffer(buf(0))
        _zero_accumulator(s, buf(0), acc_sh)

        base = pl.multiple_of(w * CHUNKS_PER_TILE, 8)
        pltpu.sync_copy(dst_hbm.at[pl.ds(base, CHUNKS_PER_TILE)], didx_v)
        plsc.subcore_barrier()

        def load_src_group(g):
            gb = pl.multiple_of(w * CHUNKS_PER_TILE + g * GROUP, GROUP)
            pltpu.sync_copy(src_hbm.at[pl.ds(gb, GROUP)], sidx_v)

        def gather(t, k, b):
            del t
            return pltpu.async_copy(u_hbm.at[sidx_v.at[k]], buf(b), gsem[b])

        def scatter(t, b):
            return pltpu.async_copy(buf(b), acc_sh.at[didx_v.at[t]], ssem[b], add=True)

        def wait_gather(b):
            pltpu.make_async_copy(u_hbm.at[sidx_v.at[0]], buf(b), gsem[b]).wait()

        def wait_scatter(t, b):
            pltpu.make_async_copy(buf(b), acc_sh.at[didx_v.at[t]], ssem[b]).wait()

        load_src_group(0)
        for k in range(GROUP):
            b = k % NBUF
            if k >= 2:
                wait_scatter(k - 2, b)
            gather(k, k, b)
            if k >= 1:
                ob = (k - 1) % NBUF
                wait_gather(ob)
                scatter(k - 1, ob)

        def body(g, carry):
            t0 = g * GROUP
            lb = (GROUP - 1) % NBUF
            wait_gather(lb)
            scatter(t0 - 1, lb)
            load_src_group(g)
            for k in range(GROUP):
                b = k % NBUF
                wait_scatter(t0 + k - 2, b)
                gather(t0 + k, k, b)
                if k >= 1:
                    ob = (k - 1) % NBUF
                    wait_gather(ob)
                    scatter(t0 + k - 1, ob)
            return carry

        lax.fori_loop(1, GROUPS_PER_TILE, body, 0)

        last_t = CHUNKS_PER_TILE - 1
        lb = (GROUP - 1) % NBUF
        wait_gather(lb)
        scatter(last_t, lb)
        wait_scatter(last_t - 1, (last_t - 1) % NBUF)
        wait_scatter(last_t, lb)

        plsc.subcore_barrier()
        _copy_out_accumulator(c, s, acc_sh, out_hbm)

    return scatter_kernel(u, src2d, dst2d)


_BR = 1024
_GRID = N_PAD // _BR


def _part_spec():
    return pl.BlockSpec((NUM_CORES, _BR, D), lambda i: (0, i, 0))


def _d_from_deg(deg_ref):
    return lax.rsqrt(deg_ref[0] + deg_ref[1] + 1.0)


def _tc_layer_in(x, W, degp):
    def body(deg_ref, x_ref, w_ref, u_ref):
        d = _d_from_deg(deg_ref)
        u_ref[...] = (
            jnp.dot(x_ref[...], w_ref[...], preferred_element_type=jnp.float32) * d
        )

    return pl.pallas_call(
        body,
        grid=(_GRID,),
        in_specs=[
            _part_spec(),
            pl.BlockSpec((_BR, D), lambda i: (i, 0)),
            pl.BlockSpec((D, D), lambda i: (0, 0)),
        ],
        out_specs=pl.BlockSpec((_BR, D), lambda i: (i, 0)),
        out_shape=jax.ShapeDtypeStruct((N_PAD, D), jnp.float32),
    )(degp, x, W)


def _tc_mid(acc, u1, b1, a, W2, degp):
    def body(deg_ref, acc_ref, u1_ref, b1_ref, a_ref, w2_ref, u2_ref):
        d = _d_from_deg(deg_ref)
        t = (acc_ref[0] + acc_ref[1] + u1_ref[...]) * d + b1_ref[...]
        t = jnp.where(t >= 0.0, t, a_ref[...] * t)
        u2_ref[...] = (
            jnp.dot(t, w2_ref[...], preferred_element_type=jnp.float32) * d
        )

    return pl.pallas_call(
        body,
        grid=(_GRID,),
        in_specs=[
            _part_spec(),
            _part_spec(),
            pl.BlockSpec((_BR, D), lambda i: (i, 0)),
            pl.BlockSpec((1, D), lambda i: (0, 0)),
            pl.BlockSpec((1, D), lambda i: (0, 0)),
            pl.BlockSpec((D, D), lambda i: (0, 0)),
        ],
        out_specs=pl.BlockSpec((_BR, D), lambda i: (i, 0)),
        out_shape=jax.ShapeDtypeStruct((N_PAD, D), jnp.float32),
    )(degp, acc, u1, b1, a, W2)


def _tc_final(acc, u2, b2, a, degp):
    def body(deg_ref, acc_ref, u2_ref, b2_ref, a_ref, o_ref):
        d = _d_from_deg(deg_ref)
        t = (acc_ref[0] + acc_ref[1] + u2_ref[...]) * d + b2_ref[...]
        o_ref[...] = jnp.where(t >= 0.0, t, a_ref[...] * t)

    return pl.pallas_call(
        body,
        grid=(_GRID,),
        in_specs=[
            _part_spec(),
            _part_spec(),
            pl.BlockSpec((_BR, D), lambda i: (i, 0)),
            pl.BlockSpec((1, D), lambda i: (0, 0)),
            pl.BlockSpec((1, D), lambda i: (0, 0)),
        ],
        out_specs=pl.BlockSpec((_BR, D), lambda i: (i, 0)),
        out_shape=jax.ShapeDtypeStruct((N_PAD, D), jnp.float32),
    )(degp, acc, u2, b2, a)


def kernel(x, edge_index, W1, b1, W2, b2, a):
    src = edge_index[0].astype(jnp.int32)
    dst = edge_index[1].astype(jnp.int32)
    n_extra = N_EDGES_PAD - N_EDGES
    src_pad = jnp.arange(n_extra, dtype=jnp.int32) % N_NODES
    dst_pad = N_NODES + jnp.arange(n_extra, dtype=jnp.int32) % (N_PAD - N_NODES)
    src2d = jnp.concatenate([src, src_pad]).reshape(N_CHUNKS_PAD, CHUNK)
    dst2d = jnp.concatenate([dst, dst_pad]).reshape(N_CHUNKS_PAD, CHUNK)
    xp = jnp.pad(x, ((0, N_PAD - N_NODES), (0, 0)))
    b1r = b1.reshape(1, D)
    b2r = b2.reshape(1, D)
    ar = a.reshape(1, D)

    degp = _sc_degree(dst2d)
    u1 = _tc_layer_in(xp, W1, degp)
    acc1 = _sc_scatter(u1, src2d, dst2d)
    u2 = _tc_mid(acc1, u1, b1r, ar, W2, degp)
    acc2 = _sc_scatter(u2, src2d, dst2d)
    return _tc_final(acc2, u2, b2r, ar, degp)[:N_NODES]
